# Initial kernel scaffold; baseline (speedup 1.0000x reference)
import functools

import jax
import jax.numpy as jnp
from jax import lax
from jax.experimental import pallas as pl
from jax.experimental.pallas import tpu as pltpu
from jax.experimental.pallas import tpu_sc as plsc

_HIST_LEN = 200
_EMBED_DIM = 16
_LBL_W = 8
_LAST3_COL = 5
_ROW_MUL = _HIST_LEN // _LBL_W
_NC, _NS = 2, 16
_NW = _NC * _NS
_CHUNK = 128
_LANES = 16


def _body(labels8, uids_hbm, emb_hbm, out_hbm,
          uids_v, idx1_v, rows8_v, idx2_v, emb_v, sem, bpw):
    wid = lax.axis_index("s") * _NC + lax.axis_index("c")
    base = wid * bpw

    pltpu.sync_copy(uids_hbm.at[pl.ds(base, bpw)], uids_v)

    for k in range(bpw // _LANES):
        p = k * _LANES
        u = uids_v[pl.ds(p, _LANES)]
        idx1_v[p // _CHUNK, pl.ds(p % _CHUNK, _LANES)] = u * _ROW_MUL + (_ROW_MUL - 1)

    n1 = bpw // _CHUNK
    cps = [pltpu.async_copy(labels8.at[idx1_v.at[i]],
                            rows8_v.at[pl.ds(i * _CHUNK, _CHUNK)], sem)
           for i in range(n1)]
    for cp in cps:
        cp.wait()

    for k in range(3 * bpw // _LANES):
        p = k * _LANES
        pos = lax.iota(jnp.int32, _LANES) + p
        bvec = pos // 3
        cvec = pos % 3 + _LAST3_COL
        ids = plsc.load_gather(rows8_v, [bvec, cvec])
        idx2_v[p // _CHUNK, pl.ds(p % _CHUNK, _LANES)] = ids

    n2 = 3 * bpw // _CHUNK
    cps = [pltpu.async_copy(emb_hbm.at[idx2_v.at[i]],
                            emb_v.at[pl.ds(i * _CHUNK, _CHUNK)], sem)
           for i in range(n2)]
    for cp in cps:
        cp.wait()

    pltpu.sync_copy(emb_v, out_hbm.at[pl.ds(3 * base, 3 * bpw)])


@jax.jit
def kernel(uids, train_labels, item_embeddings):
    batch = uids.shape[0]
    dim = item_embeddings.shape[1]
    bpw = batch // _NW
    labels8 = train_labels.reshape(-1, _LBL_W)

    run = pl.kernel(
        functools.partial(_body, bpw=bpw),
        out_type=jax.ShapeDtypeStruct((3 * batch, dim), jnp.float32),
        mesh=plsc.VectorSubcoreMesh(core_axis_name="c", subcore_axis_name="s"),
        scratch_types=[
            pltpu.VMEM((bpw,), jnp.int32),
            pltpu.VMEM((bpw // _CHUNK, _CHUNK), jnp.int32),
            pltpu.VMEM((bpw, _LBL_W), jnp.int32),
            pltpu.VMEM((3 * bpw // _CHUNK, _CHUNK), jnp.int32),
            pltpu.VMEM((3 * bpw, dim), jnp.float32),
            pltpu.SemaphoreType.DMA,
        ],
    )
    out = run(labels8, uids, item_embeddings)
    return out.reshape(batch, 3 * dim)


# speedup vs baseline: 1.2622x; 1.2622x over previous
import functools

import jax
import jax.numpy as jnp
from jax import lax
from jax.experimental import pallas as pl
from jax.experimental.pallas import tpu as pltpu
from jax.experimental.pallas import tpu_sc as plsc

_NUM_USERS = 100000
_NC, _NS = 2, 16
_NW = _NC * _NS
_CHUNK = 128
_L = 16


def _body(lab3_hbm, uids_hbm, emb_hbm, out_hbm,
          uids_v, idx1_v, ids_v, idx2_v, emb_v, sem, bpw):
    wid = lax.axis_index("s") * _NC + lax.axis_index("c")
    base = wid * bpw
    n3 = 3 * bpw

    pltpu.sync_copy(uids_hbm.at[pl.ds(base, bpw)], uids_v)

    for j in range(3):
        for k in range(bpw // _L):
            p = j * bpw + k * _L
            u = uids_v[pl.ds(k * _L, _L)]
            idx1_v[p // _CHUNK, pl.ds(p % _CHUNK, _L)] = u + j * _NUM_USERS

    cps = [pltpu.async_copy(lab3_hbm.at[idx1_v.at[i]],
                            ids_v.at[i], sem)
           for i in range(n3 // _CHUNK)]
    for cp in cps:
        cp.wait()

    three = jnp.full((_L,), 3, jnp.int32)
    iota = lax.iota(jnp.int32, _L)
    for k in range(n3 // _L):
        p = k * _L
        pos = iota + p
        b = lax.div(pos, three)
        j = lax.rem(pos, three)
        q = j * bpw + b
        qr = lax.shift_right_logical(q, 7)
        qc = lax.bitwise_and(q, _CHUNK - 1)
        idx2_v[p // _CHUNK, pl.ds(p % _CHUNK, _L)] = plsc.load_gather(
            ids_v, [qr, qc])

    cps = [pltpu.async_copy(emb_hbm.at[idx2_v.at[i]],
                            emb_v.at[pl.ds(i * _CHUNK, _CHUNK)], sem)
           for i in range(n3 // _CHUNK)]
    for cp in cps:
        cp.wait()

    pltpu.sync_copy(emb_v, out_hbm.at[pl.ds(3 * base, n3)])


@jax.jit
def kernel(uids, train_labels, item_embeddings):
    batch = uids.shape[0]
    dim = item_embeddings.shape[1]
    hist = train_labels.shape[1]
    bpw = batch // _NW
    lab3 = train_labels.T[hist - 3:hist].reshape(-1)

    run = pl.kernel(
        functools.partial(_body, bpw=bpw),
        out_type=jax.ShapeDtypeStruct((3 * batch, dim), jnp.float32),
        mesh=plsc.VectorSubcoreMesh(core_axis_name="c", subcore_axis_name="s"),
        compiler_params=pltpu.CompilerParams(
            needs_layout_passes=False, use_tc_tiling_on_sc=False),
        scratch_types=[
            pltpu.VMEM((bpw,), jnp.int32),
            pltpu.VMEM((3 * bpw // _CHUNK, _CHUNK), jnp.int32),
            pltpu.VMEM((3 * bpw // _CHUNK, _CHUNK), jnp.int32),
            pltpu.VMEM((3 * bpw // _CHUNK, _CHUNK), jnp.int32),
            pltpu.VMEM((3 * bpw, dim), jnp.float32),
            pltpu.SemaphoreType.DMA,
        ],
    )
    out = run(lab3, uids, item_embeddings)
    return out.reshape(batch, 3 * dim)
